# baseline (speedup 1.0000x reference)
import functools

import jax
import jax.numpy as jnp
from jax import lax
from jax.experimental import pallas as pl
from jax.experimental.pallas import tpu as pltpu
from jax.experimental.pallas import tpu_sc as plsc

B, L, D = 4096, 50, 64
IV, UV = 1000000, 100000
NC, NS = 2, 16
NW = NC * NS
CB = B // NW
LANES = 16

IB = IV // 128
UB = UV // 128
ITS = (IV // 256) * 256
UTS = (UV // 256) * 256
ITL = (IV - ITS) // 2
UTL = (UV - UTS) // 2


def _repack(item_t, ui_t, tail_i, tail_u):
    mesh = plsc.VectorSubcoreMesh(core_axis_name="c", subcore_axis_name="s")

    @functools.partial(
        pl.kernel,
        out_type=(
            jax.ShapeDtypeStruct((IV // 2, 128), jnp.float32),
            jax.ShapeDtypeStruct((UV // 2, 128), jnp.float32),
        ),
        mesh=mesh,
        compiler_params=pltpu.CompilerParams(needs_layout_passes=False,
                                             use_tc_tiling_on_sc=True),
        scratch_types=[
            pltpu.VMEM((D, 512), jnp.float32),
            pltpu.VMEM((D, 512), jnp.float32),
            pltpu.VMEM((128, 128), jnp.float32),
            pltpu.VMEM((128, 128), jnp.float32),
            pltpu.SemaphoreType.DMA,
            pltpu.SemaphoreType.DMA,
            pltpu.SemaphoreType.DMA,
            pltpu.SemaphoreType.DMA,
        ],
    )
    def k(src_i, src_u, tl_i, tl_u, dst_i, dst_u,
          blk_a, blk_b, lin_a, lin_b, sia, sib, soa, sob):
        wid = lax.axis_index("s") * NC + lax.axis_index("c")
        iota = lax.iota(jnp.int32, LANES)
        rowvs = [[lax.shift_right_logical(h * 256 + cg * LANES + iota, 1)
                  - h * 128 for cg in range(256 // LANES)] for h in range(2)]
        colvs = [((cg * LANES + iota) & 1) * D for cg in range(256 // LANES)]

        def transpose(blk, h, lin):
            def body(d, _):
                for cg in range(256 // LANES):
                    v = blk[d, pl.ds(h * 256 + cg * LANES, LANES)]
                    plsc.store_scatter(lin, [rowvs[h][cg], colvs[cg] + d], v)
                return 0
            lax.fori_loop(0, D, body, 0)

        def sweep(src, dst, nslab):
            nb = (nslab - wid + NW - 1) // NW

            def kb_of(t):
                tc = jnp.minimum(t, nb - 1)
                return pl.multiple_of((wid + tc * NW) * 512, 128)

            def issue_in(t, blk, sem):
                kb = kb_of(t)
                for r in range(D // 8):
                    pltpu.async_copy(
                        src.at[pl.ds(r * 8, 8), pl.ds(kb, 512)],
                        blk.at[pl.ds(r * 8, 8), :], sem)

            def wait_in(t, blk, sem):
                kb = kb_of(t)
                for r in range(D // 8):
                    pltpu.make_async_copy(
                        src.at[pl.ds(r * 8, 8), pl.ds(kb, 512)],
                        blk.at[pl.ds(r * 8, 8), :], sem).wait()

            def lb_of(t, h):
                return pl.multiple_of(kb_of(t) // 2 + h * 128, 64)

            def issue_out(t, h, lin, sem):
                return pltpu.async_copy(
                    lin, dst.at[pl.ds(lb_of(t, h), 128), :], sem)

            def wait_out(t, h, lin, sem):
                pltpu.make_async_copy(
                    lin, dst.at[pl.ds(lb_of(t, h), 128), :], sem).wait()

            issue_in(jnp.int32(0), blk_a, sia)

            def body(g2, _):
                g = g2 * 2
                issue_in(g + 1, blk_b, sib)
                wait_in(g, blk_a, sia)

                @pl.when(g2 > 0)
                def _():
                    wait_out(g - 1, 0, lin_a, soa)
                    wait_out(g - 1, 1, lin_b, sob)

                transpose(blk_a, 0, lin_a)
                issue_out(g, 0, lin_a, soa)
                transpose(blk_a, 1, lin_b)
                issue_out(g, 1, lin_b, sob)
                issue_in(g + 2, blk_a, sia)
                wait_in(g + 1, blk_b, sib)
                wait_out(g, 0, lin_a, soa)
                wait_out(g, 1, lin_b, sob)
                transpose(blk_b, 0, lin_a)
                issue_out(g + 1, 0, lin_a, soa)
                transpose(blk_b, 1, lin_b)
                issue_out(g + 1, 1, lin_b, sob)
                return 0

            nb2 = (nb + 1) // 2
            lax.fori_loop(0, nb2, body, 0)
            wait_in(jnp.int32(nb - 1), blk_a, sia)
            wait_out(jnp.int32(nb - 1), 0, lin_a, soa)
            wait_out(jnp.int32(nb - 1), 1, lin_b, sob)

        sweep(src_i, dst_i, IV // 512)
        sweep(src_u, dst_u, UV // 512)

        @pl.when(wid == NW - 1)
        def _():
            pltpu.sync_copy(tl_i, lin_a.at[pl.ds(0, ITL), :])
            pltpu.sync_copy(lin_a.at[pl.ds(0, ITL), :],
                            dst_i.at[pl.ds(ITS // 2, ITL), :])

        @pl.when(wid == NW - 2)
        def _():
            pltpu.sync_copy(tl_u, lin_b.at[pl.ds(0, UTL), :])
            pltpu.sync_copy(lin_b.at[pl.ds(0, UTL), :],
                            dst_u.at[pl.ds(UTS // 2, UTL), :])

    return k(item_t, ui_t, tail_i, tail_u)


def _scores(seq_t, uidx, pidx, nidx, dpos, dneg, item_pair, ui_pair):
    mesh = plsc.VectorSubcoreMesh(core_axis_name="c", subcore_axis_name="s")

    @functools.partial(
        pl.kernel,
        out_type=(
            jax.ShapeDtypeStruct((B,), jnp.float32),
            jax.ShapeDtypeStruct((B,), jnp.float32),
        ),
        mesh=mesh,
        compiler_params=pltpu.CompilerParams(needs_layout_passes=False,
                                             use_tc_tiling_on_sc=True),
        scratch_types=[
            pltpu.VMEM((L, CB), jnp.int32),
            pltpu.VMEM((L * CB,), jnp.int32),
            pltpu.VMEM((CB,), jnp.int32),
            pltpu.VMEM((CB,), jnp.int32),
            pltpu.VMEM((CB,), jnp.int32),
            pltpu.VMEM((CB,), jnp.int32),
            pltpu.VMEM((CB,), jnp.float32),
            pltpu.VMEM((CB,), jnp.float32),
            pltpu.VMEM((2 * CB, 128), jnp.float32),
            pltpu.VMEM((2 * CB, 128), jnp.float32),
            pltpu.VMEM((CB * D,), jnp.float32),
            pltpu.VMEM((CB * D,), jnp.float32),
            pltpu.VMEM((CB * D,), jnp.float32),
            pltpu.VMEM((CB * D,), jnp.float32),
            pltpu.VMEM((CB,), jnp.float32),
            pltpu.VMEM((CB,), jnp.float32),
            pltpu.SemaphoreType.DMA,
            pltpu.SemaphoreType.DMA,
        ],
    )
    def k(seq_hbm, uidx_hbm, pidx_hbm, nidx_hbm, dpos_hbm, dneg_hbm,
          item_hbm, ui_hbm, opos_hbm, oneg_hbm,
          sidx_v, sline_v, uidx_v, pidx_v, nidx_v, tline_v, dpos_v, dneg_v,
          stage_a, stage_b, ui_rows, pos_rows, neg_rows, acc_v,
          opos_v, oneg_v, sem_a, sem_b):
        wid = lax.axis_index("s") * NC + lax.axis_index("c")
        base = wid * CB

        pltpu.sync_copy(seq_hbm.at[:, pl.ds(base, CB)], sidx_v)
        pltpu.sync_copy(uidx_hbm.at[pl.ds(base, CB)], uidx_v)
        pltpu.sync_copy(pidx_hbm.at[pl.ds(base, CB)], pidx_v)
        pltpu.sync_copy(nidx_hbm.at[pl.ds(base, CB)], nidx_v)
        pltpu.sync_copy(dpos_hbm.at[pl.ds(base, CB)], dpos_v)
        pltpu.sync_copy(dneg_hbm.at[pl.ds(base, CB)], dneg_v)

        def lbody(i, _):
            l = i // (CB // LANES)
            gidx = i % (CB // LANES)
            s = sidx_v[l, pl.ds(gidx * LANES, LANES)]
            sline_v[pl.ds(i * LANES, LANES)] = lax.shift_right_logical(s, 1)
            return 0
        lax.fori_loop(0, L * CB // LANES, lbody, 0, unroll=8)

        for (idxv, rows) in ((uidx_v, ui_rows), (pidx_v, pos_rows),
                             (nidx_v, neg_rows)):
            tbl = ui_hbm if rows is ui_rows else item_hbm

            def g2body(gidx, _, idxv=idxv):
                s = idxv[pl.ds(gidx * LANES, LANES)]
                tline_v[pl.ds(gidx * LANES, LANES)] = (
                    lax.shift_right_logical(s, 1))
                return 0
            lax.fori_loop(0, CB // LANES, g2body, 0)
            pltpu.async_copy(tbl.at[tline_v], stage_a.at[pl.ds(0, CB)],
                             sem_a).wait()

            def cpbody(bg, _, idxv=idxv, rows=rows):
                pv = jnp.asarray(
                    idxv[pl.ds(bg * LANES, LANES)] & 1, jnp.float32)
                for j in range(LANES):
                    bb = bg * LANES + j
                    p = pv[j]
                    for kk in range(D // LANES):
                        s0 = stage_a[bb, pl.ds(kk * LANES, LANES)]
                        s1 = stage_a[bb, pl.ds(D + kk * LANES, LANES)]
                        rows[pl.ds(bb * D + kk * LANES, LANES)] = (
                            s0 * (1.0 - p) + s1 * p)
                return 0

            lax.fori_loop(0, CB // LANES, cpbody, 0)

        def zbody(i, _):
            acc_v[pl.ds(i * LANES, LANES)] = jnp.zeros((LANES,), jnp.float32)
            return 0
        lax.fori_loop(0, CB * D // LANES, zbody, 0)

        NSTEP = L // 2

        def issue(t, buf, sem):
            return pltpu.async_copy(
                item_hbm.at[sline_v.at[pl.ds(t * 2 * CB, 2 * CB)]], buf, sem)

        def wait_g(t, buf, sem):
            pltpu.make_async_copy(
                item_hbm.at[sline_v.at[pl.ds(t * 2 * CB, 2 * CB)]],
                buf, sem).wait()

        def process(t, buf):
            def pbody(bg, _):
                for half in range(2):
                    l = t * 2 + half
                    sv = sidx_v[l, pl.ds(bg * LANES, LANES)]
                    pv = jnp.asarray(sv & 1, jnp.float32)
                    mv = jnp.where(sv == 0, jnp.float32(0), jnp.float32(1))
                    w1v = pv * mv
                    w0v = (1.0 - pv) * mv
                    for j in range(LANES):
                        bb = bg * LANES + j
                        sb = half * CB + bb
                        w0 = w0v[j]
                        w1 = w1v[j]
                        for kk in range(D // LANES):
                            s0 = buf[sb, pl.ds(kk * LANES, LANES)]
                            s1 = buf[sb, pl.ds(D + kk * LANES, LANES)]
                            dst = pl.ds(bb * D + kk * LANES, LANES)
                            acc_v[dst] = acc_v[dst] + s0 * w0 + s1 * w1
                return 0

            lax.fori_loop(0, CB // LANES, pbody, 0)

        issue(jnp.int32(0), stage_a, sem_a)

        def body(g2, _):
            g = g2 * 2
            issue(g + 1, stage_b, sem_b)
            wait_g(g, stage_a, sem_a)
            process(g, stage_a)
            issue(g + 2, stage_a, sem_a)
            wait_g(g + 1, stage_b, sem_b)
            process(g + 1, stage_b)
            return 0

        lax.fori_loop(0, (NSTEP - 1) // 2, body, 0)
        wait_g(jnp.int32(NSTEP - 1), stage_a, sem_a)
        process(jnp.int32(NSTEP - 1), stage_a)

        iota = lax.iota(jnp.int32, LANES)
        for gg in range(CB // LANES):
            rbase = gg * LANES

            def cbody(l, n0):
                sv = sidx_v[l, pl.ds(rbase, LANES)]
                return n0 + jnp.where(sv == 0, jnp.float32(1), jnp.float32(0))

            n0 = lax.fori_loop(0, L, cbody, jnp.zeros((LANES,), jnp.float32))
            inv = jnp.float32(1) / (jnp.float32(L) - n0 + jnp.float32(1e-9))
            flat = (rbase + iota) * D

            def dbody(d, carry):
                ps, ns = carry
                fd = flat + d
                a = plsc.load_gather(acc_v, [fd])
                u = plsc.load_gather(ui_rows, [fd]) + a * inv
                p = plsc.load_gather(pos_rows, [fd])
                nn = plsc.load_gather(neg_rows, [fd])
                return (ps + u * p, ns + u * nn)

            z = jnp.zeros((LANES,), jnp.float32)
            ps, ns = lax.fori_loop(0, D, dbody, (z, z))
            opos_v[pl.ds(rbase, LANES)] = ps - dpos_v[pl.ds(rbase, LANES)]
            oneg_v[pl.ds(rbase, LANES)] = ns - dneg_v[pl.ds(rbase, LANES)]

        pltpu.sync_copy(opos_v, opos_hbm.at[pl.ds(base, CB)])
        pltpu.sync_copy(oneg_v, oneg_hbm.at[pl.ds(base, CB)])

    return k(seq_t, uidx, pidx, nidx, dpos, dneg, item_pair, ui_pair)


def kernel(user_inputs, seq_inputs, pos_inputs, neg_inputs, distance_pos,
           distance_neg, ui_table, item_table):
    seq_t = seq_inputs.T.astype(jnp.int32)
    uidx = user_inputs.reshape(-1).astype(jnp.int32)
    pidx = pos_inputs.reshape(-1).astype(jnp.int32)
    nidx = neg_inputs.reshape(-1).astype(jnp.int32)
    dpos = distance_pos.reshape(-1).astype(jnp.float32)
    dneg = distance_neg.reshape(-1).astype(jnp.float32)
    tail_i = item_table[ITS:, :].reshape(ITL, 128)
    tail_u = ui_table[UTS:, :].reshape(UTL, 128)
    item_pair, ui_pair = _repack(item_table.T, ui_table.T, tail_i, tail_u)
    pos_s, neg_s = _scores(seq_t, uidx, pidx, nidx, dpos, dneg,
                           item_pair, ui_pair)
    return (pos_s.reshape(B, 1), neg_s.reshape(B, 1))


# speedup vs baseline: 2.3823x; 2.3823x over previous
import functools

import jax
import jax.numpy as jnp
from jax import lax
from jax.experimental import pallas as pl
from jax.experimental.pallas import tpu as pltpu
from jax.experimental.pallas import tpu_sc as plsc

B, L, D = 4096, 50, 64
NC, NS = 2, 16
NW = NC * NS
CB = B // NW
SUB = 8
NSUB = CB // SUB
ROWS = SUB * L
LANES = 16


def _scores(sidx, uidx, pidx, nidx, dpos, dneg, ui_table, item_table):
    mesh = plsc.VectorSubcoreMesh(core_axis_name="c", subcore_axis_name="s")

    @functools.partial(
        pl.kernel,
        out_type=(
            jax.ShapeDtypeStruct((B,), jnp.float32),
            jax.ShapeDtypeStruct((B,), jnp.float32),
        ),
        mesh=mesh,
        compiler_params=pltpu.CompilerParams(needs_layout_passes=False,
                                             use_tc_tiling_on_sc=False),
        scratch_types=[
            pltpu.VMEM((CB * L,), jnp.int32),
            pltpu.VMEM((CB,), jnp.int32),
            pltpu.VMEM((CB,), jnp.int32),
            pltpu.VMEM((CB,), jnp.int32),
            pltpu.VMEM((CB,), jnp.float32),
            pltpu.VMEM((CB,), jnp.float32),
            pltpu.VMEM((ROWS, D), jnp.float32),
            pltpu.VMEM((ROWS, D), jnp.float32),
            pltpu.VMEM((CB, D), jnp.float32),
            pltpu.VMEM((CB, D), jnp.float32),
            pltpu.VMEM((CB, D), jnp.float32),
            pltpu.VMEM((CB, D), jnp.float32),
            pltpu.VMEM((CB,), jnp.float32),
            pltpu.VMEM((CB,), jnp.float32),
            pltpu.VMEM((1, D), jnp.float32),
            pltpu.SemaphoreType.DMA,
            pltpu.SemaphoreType.DMA,
            pltpu.SemaphoreType.DMA,
            pltpu.SemaphoreType.DMA,
            pltpu.SemaphoreType.DMA,
        ],
    )
    def k(sidx_hbm, uidx_hbm, pidx_hbm, nidx_hbm, dpos_hbm, dneg_hbm,
          ui_hbm, item_hbm, opos_hbm, oneg_hbm,
          sidx_v, uidx_v, pidx_v, nidx_v, dpos_v, dneg_v,
          stage0, stage1, ui_rows, pos_rows, neg_rows, acc_v,
          opos_v, oneg_v, t0_v, sem_u, sem_p, sem_n, sem_s0, sem_s1):
        wid = lax.axis_index("s") * NC + lax.axis_index("c")
        base = wid * CB

        pltpu.sync_copy(sidx_hbm.at[pl.ds(base * L, CB * L)], sidx_v)
        pltpu.sync_copy(uidx_hbm.at[pl.ds(base, CB)], uidx_v)
        pltpu.sync_copy(pidx_hbm.at[pl.ds(base, CB)], pidx_v)
        pltpu.sync_copy(nidx_hbm.at[pl.ds(base, CB)], nidx_v)
        pltpu.sync_copy(dpos_hbm.at[pl.ds(base, CB)], dpos_v)
        pltpu.sync_copy(dneg_hbm.at[pl.ds(base, CB)], dneg_v)
        pltpu.sync_copy(item_hbm.at[pl.ds(0, 1)], t0_v)

        cu = pltpu.async_copy(ui_hbm.at[uidx_v], ui_rows, sem_u)
        cp = pltpu.async_copy(item_hbm.at[pidx_v], pos_rows, sem_p)
        cn = pltpu.async_copy(item_hbm.at[nidx_v], neg_rows, sem_n)

        stages = (stage0, stage1)
        sems = (sem_s0, sem_s1)
        handles = [
            pltpu.async_copy(item_hbm.at[sidx_v.at[pl.ds(0, ROWS)]],
                             stage0, sem_s0),
            None,
        ]
        for s in range(NSUB):
            buf = s % 2
            if s + 1 < NSUB:
                nb = (s + 1) % 2
                handles[nb] = pltpu.async_copy(
                    item_hbm.at[sidx_v.at[pl.ds((s + 1) * ROWS, ROWS)]],
                    stages[nb], sems[nb])
            handles[buf].wait()
            st = stages[buf]

            def outer(bl, _, s=s, st=st):
                b = s * SUB + bl

                def inner(l, carry, bl=bl, st=st):
                    a0, a1, a2, a3 = carry
                    r = bl * L + l
                    return (a0 + st[r, pl.ds(0, LANES)],
                            a1 + st[r, pl.ds(LANES, LANES)],
                            a2 + st[r, pl.ds(2 * LANES, LANES)],
                            a3 + st[r, pl.ds(3 * LANES, LANES)])

                z = jnp.zeros((LANES,), jnp.float32)
                a0, a1, a2, a3 = lax.fori_loop(0, L, inner, (z, z, z, z))
                acc_v[b, pl.ds(0, LANES)] = a0
                acc_v[b, pl.ds(LANES, LANES)] = a1
                acc_v[b, pl.ds(2 * LANES, LANES)] = a2
                acc_v[b, pl.ds(3 * LANES, LANES)] = a3
                return 0

            lax.fori_loop(0, SUB, outer, 0)

        cu.wait()
        cp.wait()
        cn.wait()

        iota = lax.iota(jnp.int32, LANES)
        for g in range(CB // LANES):
            rbase = g * LANES
            rows_idx = iota + rbase

            def cbody(l, n0):
                sv = plsc.load_gather(sidx_v, [rbase * L + iota * L + l])
                return n0 + jnp.where(sv == 0, jnp.float32(1), jnp.float32(0))

            n0 = lax.fori_loop(0, L, cbody, jnp.zeros((LANES,), jnp.float32))
            inv = jnp.float32(1) / (jnp.float32(L) - n0 + jnp.float32(1e-9))

            zrow = jnp.zeros((LANES,), jnp.int32)

            def dbody(d, carry):
                ps, ns = carry
                dcol = jnp.full((LANES,), d, jnp.int32)
                t0c = plsc.load_gather(t0_v, [zrow, dcol])
                a = plsc.load_gather(acc_v, [rows_idx, dcol]) - n0 * t0c
                u = plsc.load_gather(ui_rows, [rows_idx, dcol]) + a * inv
                p = plsc.load_gather(pos_rows, [rows_idx, dcol])
                nn = plsc.load_gather(neg_rows, [rows_idx, dcol])
                return (ps + u * p, ns + u * nn)

            z = jnp.zeros((LANES,), jnp.float32)
            ps, ns = lax.fori_loop(0, D, dbody, (z, z))
            opos_v[pl.ds(rbase, LANES)] = ps - dpos_v[pl.ds(rbase, LANES)]
            oneg_v[pl.ds(rbase, LANES)] = ns - dneg_v[pl.ds(rbase, LANES)]

        pltpu.sync_copy(opos_v, opos_hbm.at[pl.ds(base, CB)])
        pltpu.sync_copy(oneg_v, oneg_hbm.at[pl.ds(base, CB)])

    return k(sidx, uidx, pidx, nidx, dpos, dneg, ui_table, item_table)


def kernel(user_inputs, seq_inputs, pos_inputs, neg_inputs, distance_pos,
           distance_neg, ui_table, item_table):
    sidx = seq_inputs.reshape(-1).astype(jnp.int32)
    uidx = user_inputs.reshape(-1).astype(jnp.int32)
    pidx = pos_inputs.reshape(-1).astype(jnp.int32)
    nidx = neg_inputs.reshape(-1).astype(jnp.int32)
    dpos = distance_pos.reshape(-1).astype(jnp.float32)
    dneg = distance_neg.reshape(-1).astype(jnp.float32)
    pos_s, neg_s = _scores(sidx, uidx, pidx, nidx, dpos, dneg,
                           ui_table, item_table)
    return (pos_s.reshape(B, 1), neg_s.reshape(B, 1))


# speedup vs baseline: 2.3983x; 1.0067x over previous
import functools

import jax
import jax.numpy as jnp
from jax import lax
from jax.experimental import pallas as pl
from jax.experimental.pallas import tpu as pltpu
from jax.experimental.pallas import tpu_sc as plsc

B, L, D = 4096, 50, 64
NC, NS = 2, 16
NW = NC * NS
CB = B // NW
SUB = 4
NSUB = CB // SUB
ROWS = SUB * L
LANES = 16


def _scores(sidx, uidx, pidx, nidx, dpos, dneg, ui_table, item_table):
    mesh = plsc.VectorSubcoreMesh(core_axis_name="c", subcore_axis_name="s")

    @functools.partial(
        pl.kernel,
        out_type=(
            jax.ShapeDtypeStruct((B,), jnp.float32),
            jax.ShapeDtypeStruct((B,), jnp.float32),
        ),
        mesh=mesh,
        compiler_params=pltpu.CompilerParams(needs_layout_passes=False,
                                             use_tc_tiling_on_sc=False),
        scratch_types=[
            pltpu.VMEM((CB * L,), jnp.int32),
            pltpu.VMEM((CB,), jnp.int32),
            pltpu.VMEM((CB,), jnp.int32),
            pltpu.VMEM((CB,), jnp.int32),
            pltpu.VMEM((CB,), jnp.float32),
            pltpu.VMEM((CB,), jnp.float32),
            pltpu.VMEM((ROWS, D), jnp.float32),
            pltpu.VMEM((ROWS, D), jnp.float32),
            pltpu.VMEM((ROWS, D), jnp.float32),
            pltpu.VMEM((CB, D), jnp.float32),
            pltpu.VMEM((CB, D), jnp.float32),
            pltpu.VMEM((CB, D), jnp.float32),
            pltpu.VMEM((CB, D), jnp.float32),
            pltpu.VMEM((CB,), jnp.float32),
            pltpu.VMEM((CB,), jnp.float32),
            pltpu.VMEM((1, D), jnp.float32),
            pltpu.SemaphoreType.DMA,
            pltpu.SemaphoreType.DMA,
            pltpu.SemaphoreType.DMA,
            pltpu.SemaphoreType.DMA,
            pltpu.SemaphoreType.DMA,
            pltpu.SemaphoreType.DMA,
        ],
    )
    def k(sidx_hbm, uidx_hbm, pidx_hbm, nidx_hbm, dpos_hbm, dneg_hbm,
          ui_hbm, item_hbm, opos_hbm, oneg_hbm,
          sidx_v, uidx_v, pidx_v, nidx_v, dpos_v, dneg_v,
          stage0, stage1, stage2, ui_rows, pos_rows, neg_rows, acc_v,
          opos_v, oneg_v, t0_v, sem_u, sem_p, sem_n,
          sem_s0, sem_s1, sem_s2):
        wid = lax.axis_index("s") * NC + lax.axis_index("c")
        base = wid * CB

        pltpu.sync_copy(sidx_hbm.at[pl.ds(base * L, CB * L)], sidx_v)
        pltpu.sync_copy(uidx_hbm.at[pl.ds(base, CB)], uidx_v)
        pltpu.sync_copy(pidx_hbm.at[pl.ds(base, CB)], pidx_v)
        pltpu.sync_copy(nidx_hbm.at[pl.ds(base, CB)], nidx_v)
        pltpu.sync_copy(dpos_hbm.at[pl.ds(base, CB)], dpos_v)
        pltpu.sync_copy(dneg_hbm.at[pl.ds(base, CB)], dneg_v)
        pltpu.sync_copy(item_hbm.at[pl.ds(0, 1)], t0_v)

        cu = pltpu.async_copy(ui_hbm.at[uidx_v], ui_rows, sem_u)
        cp = pltpu.async_copy(item_hbm.at[pidx_v], pos_rows, sem_p)
        cn = pltpu.async_copy(item_hbm.at[nidx_v], neg_rows, sem_n)

        stages = (stage0, stage1, stage2)
        sems = (sem_s0, sem_s1, sem_s2)
        handles = [
            pltpu.async_copy(item_hbm.at[sidx_v.at[pl.ds(0, ROWS)]],
                             stage0, sem_s0),
            pltpu.async_copy(item_hbm.at[sidx_v.at[pl.ds(ROWS, ROWS)]],
                             stage1, sem_s1),
            None,
        ]
        for s in range(NSUB):
            buf = s % 3
            if s + 2 < NSUB:
                nb = (s + 2) % 3
                handles[nb] = pltpu.async_copy(
                    item_hbm.at[sidx_v.at[pl.ds((s + 2) * ROWS, ROWS)]],
                    stages[nb], sems[nb])
            handles[buf].wait()
            st = stages[buf]

            def outer(bl, _, s=s, st=st):
                b = s * SUB + bl

                def inner(l, carry, bl=bl, st=st):
                    a0, a1, a2, a3 = carry
                    r = bl * L + l
                    return (a0 + st[r, pl.ds(0, LANES)],
                            a1 + st[r, pl.ds(LANES, LANES)],
                            a2 + st[r, pl.ds(2 * LANES, LANES)],
                            a3 + st[r, pl.ds(3 * LANES, LANES)])

                z = jnp.zeros((LANES,), jnp.float32)
                a0, a1, a2, a3 = lax.fori_loop(0, L, inner, (z, z, z, z))
                acc_v[b, pl.ds(0, LANES)] = a0
                acc_v[b, pl.ds(LANES, LANES)] = a1
                acc_v[b, pl.ds(2 * LANES, LANES)] = a2
                acc_v[b, pl.ds(3 * LANES, LANES)] = a3
                return 0

            lax.fori_loop(0, SUB, outer, 0)

        cu.wait()
        cp.wait()
        cn.wait()

        iota = lax.iota(jnp.int32, LANES)
        for g in range(CB // LANES):
            rbase = g * LANES
            rows_idx = iota + rbase

            def cbody(l, n0):
                sv = plsc.load_gather(sidx_v, [rbase * L + iota * L + l])
                return n0 + jnp.where(sv == 0, jnp.float32(1), jnp.float32(0))

            n0 = lax.fori_loop(0, L, cbody, jnp.zeros((LANES,), jnp.float32))
            inv = jnp.float32(1) / (jnp.float32(L) - n0 + jnp.float32(1e-9))

            zrow = jnp.zeros((LANES,), jnp.int32)

            def dbody(d, carry):
                ps, ns = carry
                dcol = jnp.full((LANES,), d, jnp.int32)
                t0c = plsc.load_gather(t0_v, [zrow, dcol])
                a = plsc.load_gather(acc_v, [rows_idx, dcol]) - n0 * t0c
                u = plsc.load_gather(ui_rows, [rows_idx, dcol]) + a * inv
                p = plsc.load_gather(pos_rows, [rows_idx, dcol])
                nn = plsc.load_gather(neg_rows, [rows_idx, dcol])
                return (ps + u * p, ns + u * nn)

            z = jnp.zeros((LANES,), jnp.float32)
            ps, ns = lax.fori_loop(0, D, dbody, (z, z))
            opos_v[pl.ds(rbase, LANES)] = ps - dpos_v[pl.ds(rbase, LANES)]
            oneg_v[pl.ds(rbase, LANES)] = ns - dneg_v[pl.ds(rbase, LANES)]

        pltpu.sync_copy(opos_v, opos_hbm.at[pl.ds(base, CB)])
        pltpu.sync_copy(oneg_v, oneg_hbm.at[pl.ds(base, CB)])

    return k(sidx, uidx, pidx, nidx, dpos, dneg, ui_table, item_table)


def kernel(user_inputs, seq_inputs, pos_inputs, neg_inputs, distance_pos,
           distance_neg, ui_table, item_table):
    sidx = seq_inputs.reshape(-1).astype(jnp.int32)
    uidx = user_inputs.reshape(-1).astype(jnp.int32)
    pidx = pos_inputs.reshape(-1).astype(jnp.int32)
    nidx = neg_inputs.reshape(-1).astype(jnp.int32)
    dpos = distance_pos.reshape(-1).astype(jnp.float32)
    dneg = distance_neg.reshape(-1).astype(jnp.float32)
    pos_s, neg_s = _scores(sidx, uidx, pidx, nidx, dpos, dneg,
                           ui_table, item_table)
    return (pos_s.reshape(B, 1), neg_s.reshape(B, 1))
